# baseline (speedup 1.0000x reference)
import functools

import jax
import jax.numpy as jnp
import numpy as np
from jax import lax
from jax.experimental import pallas as pl
from jax.experimental.pallas import tpu as pltpu
from jax.experimental.pallas import tpu_sc as plsc

NC = 2
NS = 16
NW = NC * NS
L = 16


def _interleave_perm(d):
    perm = np.zeros(d, dtype=np.int32)
    for g in range(d // 32):
        for i in range(16):
            perm[g * 32 + 2 * i] = g * 32 + i
            perm[g * 32 + 2 * i + 1] = g * 32 + 16 + i
    return perm


def _project(u2e, base, W1, b1):
    N, D = u2e.shape
    deg_inv = 1.0 / 32.0
    perm = _interleave_perm(D)
    Wa = W1[0:D][:, perm]
    Wb = W1[D:2 * D][:, perm]
    Wc = W1[2 * D:3 * D][:, perm]
    b1_2d = b1[perm].reshape(1, D)

    BLK = 2000
    assert N % BLK == 0

    def body(u_ref, c_ref, wa_ref, wb_ref, wc_ref, b1_ref, q_ref, p_ref):
        u = u_ref[...]
        q_ref[...] = (
            jnp.dot(u, wa_ref[...], preferred_element_type=jnp.float32,
                    precision=lax.Precision.DEFAULT)
            + jnp.dot(c_ref[...], wc_ref[...], preferred_element_type=jnp.float32,
                      precision=lax.Precision.DEFAULT)
            + b1_ref[...]
        ).astype(jnp.bfloat16)
        p_ref[...] = (jnp.dot(u, wb_ref[...], preferred_element_type=jnp.float32,
                              precision=lax.Precision.DEFAULT)
                      * deg_inv).astype(jnp.bfloat16)

    grid = (N // BLK,)
    return pl.pallas_call(
        body,
        grid=grid,
        in_specs=[
            pl.BlockSpec((BLK, D), lambda i: (i, 0)),
            pl.BlockSpec((BLK, D), lambda i: (i, 0)),
            pl.BlockSpec((D, D), lambda i: (0, 0)),
            pl.BlockSpec((D, D), lambda i: (0, 0)),
            pl.BlockSpec((D, D), lambda i: (0, 0)),
            pl.BlockSpec((1, D), lambda i: (0, 0)),
        ],
        out_specs=[
            pl.BlockSpec((BLK, D), lambda i: (i, 0)),
            pl.BlockSpec((BLK, D), lambda i: (i, 0)),
        ],
        out_shape=[
            jax.ShapeDtypeStruct((N, D), jnp.bfloat16),
            jax.ShapeDtypeStruct((N, D), jnp.bfloat16),
        ],
    )(u2e, base, Wa, Wb, Wc, b1_2d)


def _sc_gather_agg(nodes, neighbors, q_tab, p_tab):
    B, = nodes.shape
    N, DEG = neighbors.shape
    D = q_tab.shape[1]
    BPW = B // NW
    CH = 4
    NCHUNK = BPW // CH
    mesh = plsc.VectorSubcoreMesh(core_axis_name="c", subcore_axis_name="s")

    @functools.partial(
        pl.kernel,
        mesh=mesh,
        out_type=jax.ShapeDtypeStruct((B, D), jnp.float32),
        compiler_params=pltpu.CompilerParams(use_tc_tiling_on_sc=False,
                                             needs_layout_passes=False),
        scratch_types=[
            pltpu.VMEM((BPW,), jnp.int32),
            pltpu.VMEM((BPW, DEG), jnp.int32),
            pltpu.VMEM((BPW * DEG,), jnp.int32),
            pltpu.VMEM((BPW, D), jnp.bfloat16),
            pltpu.VMEM((CH * DEG, D), jnp.bfloat16),
            pltpu.VMEM((CH * DEG, D), jnp.bfloat16),
            pltpu.VMEM((BPW, D), jnp.float32),
            pltpu.SemaphoreType.DMA,
            pltpu.SemaphoreType.DMA,
            pltpu.SemaphoreType.DMA,
        ],
    )
    def k(nodes_hbm, neigh_hbm, q_hbm, p_hbm, out_hbm,
          idx_v, nidx_v, flat_v, q_v, buf0, buf1, out_v, sem0, sem1, semq):
        wid = lax.axis_index("s") * NC + lax.axis_index("c")
        base = wid * BPW
        pltpu.sync_copy(nodes_hbm.at[pl.ds(base, BPW)], idx_v)
        pltpu.async_copy(neigh_hbm.at[idx_v], nidx_v, sem0).wait()
        pltpu.async_copy(q_hbm.at[idx_v], q_v, semq)

        @pl.loop(0, BPW)
        def _(i):
            for j in range(DEG // L):
                flat_v[pl.ds(i * DEG + j * L, L)] = nidx_v[i, pl.ds(j * L, L)]

        def issue(c, buf, sem):
            pltpu.async_copy(
                p_hbm.at[flat_v.at[pl.ds(c * (CH * DEG), CH * DEG)]], buf, sem)

        def drain(buf, sem):
            pltpu.make_async_copy(
                p_hbm.at[flat_v.at[pl.ds(0, CH * DEG)]], buf, sem).wait()

        def accum(c, buf):
            @pl.loop(0, CH)
            def _(rr):
                row = c * CH + rr
                for g in range(D // (2 * L)):
                    sl = pl.ds(g * 2 * L, 2 * L)
                    accs = [buf[rr * DEG + t, sl] for t in range(4)]
                    for j in range(4, DEG):
                        accs[j % 4] = accs[j % 4] + buf[rr * DEG + j, sl]
                    acc = (accs[0] + accs[1]) + (accs[2] + accs[3])
                    acc = jnp.maximum(acc + q_v[row, sl],
                                      jnp.bfloat16(0.0).astype(jnp.bfloat16))
                    lo, hi = plsc.unpack(acc, format=plsc.PackFormat.INTERLEAVED)
                    out_v[row, pl.ds(g * 2 * L, L)] = lo
                    out_v[row, pl.ds(g * 2 * L + L, L)] = hi

        issue(0, buf0, sem0)
        pltpu.make_async_copy(q_hbm.at[idx_v], q_v, semq).wait()

        @pl.loop(0, NCHUNK, step=2)
        def _(c):
            issue(c + 1, buf1, sem1)
            drain(buf0, sem0)
            accum(c, buf0)

            @pl.when(c + 2 < NCHUNK)
            def _():
                issue(c + 2, buf0, sem0)

            drain(buf1, sem1)
            accum(c + 1, buf1)

        pltpu.sync_copy(out_v, out_hbm.at[pl.ds(base, BPW)])

    return k(nodes, neighbors, q_tab, p_tab)


def kernel(nodes, neighbors, u2e_weight, base_weight, W1, b1):
    q_tab, p_tab = _project(u2e_weight, base_weight, W1, b1)
    return _sc_gather_agg(nodes, neighbors, q_tab, p_tab)


# speedup vs baseline: 1.1222x; 1.1222x over previous
import functools

import jax
import jax.numpy as jnp
import numpy as np
from jax import lax
from jax.experimental import pallas as pl
from jax.experimental.pallas import tpu as pltpu
from jax.experimental.pallas import tpu_sc as plsc

NC = 2
NS = 16
NW = NC * NS
L = 16


def _interleave_perm(d):
    perm = np.zeros(d, dtype=np.int32)
    for g in range(d // 32):
        for i in range(16):
            perm[g * 32 + 2 * i] = g * 32 + i
            perm[g * 32 + 2 * i + 1] = g * 32 + 16 + i
    return perm


def _project(u2e, base, W1, b1):
    N, D = u2e.shape
    deg_inv = 1.0 / 32.0
    Wa = W1[0:D]
    Wb = W1[D:2 * D][:, _interleave_perm(D)]
    Wc = W1[2 * D:3 * D]
    b1_2d = b1.reshape(1, D)

    BLK = 2000
    assert N % BLK == 0

    def body(u_ref, c_ref, wa_ref, wb_ref, wc_ref, b1_ref, q_ref, p_ref):
        u = u_ref[...]
        q_ref[...] = (
            jnp.dot(u, wa_ref[...], preferred_element_type=jnp.float32,
                    precision=lax.Precision.DEFAULT)
            + jnp.dot(c_ref[...], wc_ref[...], preferred_element_type=jnp.float32,
                      precision=lax.Precision.DEFAULT)
            + b1_ref[...]
        )
        p_ref[...] = (jnp.dot(u, wb_ref[...], preferred_element_type=jnp.float32,
                              precision=lax.Precision.DEFAULT)
                      * deg_inv).astype(jnp.bfloat16)

    grid = (N // BLK,)
    return pl.pallas_call(
        body,
        grid=grid,
        in_specs=[
            pl.BlockSpec((BLK, D), lambda i: (i, 0)),
            pl.BlockSpec((BLK, D), lambda i: (i, 0)),
            pl.BlockSpec((D, D), lambda i: (0, 0)),
            pl.BlockSpec((D, D), lambda i: (0, 0)),
            pl.BlockSpec((D, D), lambda i: (0, 0)),
            pl.BlockSpec((1, D), lambda i: (0, 0)),
        ],
        out_specs=[
            pl.BlockSpec((BLK, D), lambda i: (i, 0)),
            pl.BlockSpec((BLK, D), lambda i: (i, 0)),
        ],
        out_shape=[
            jax.ShapeDtypeStruct((N, D), jnp.float32),
            jax.ShapeDtypeStruct((N, D), jnp.bfloat16),
        ],
    )(u2e, base, Wa, Wb, Wc, b1_2d)


def _sc_index(nodes, neighbors):
    B, = nodes.shape
    N, DEG = neighbors.shape
    BPW = B // NW
    mesh = plsc.VectorSubcoreMesh(core_axis_name="c", subcore_axis_name="s")

    @functools.partial(
        pl.kernel,
        mesh=mesh,
        out_type=jax.ShapeDtypeStruct((B * DEG,), jnp.int32),
        compiler_params=pltpu.CompilerParams(use_tc_tiling_on_sc=False,
                                             needs_layout_passes=False),
        scratch_types=[
            pltpu.VMEM((BPW,), jnp.int32),
            pltpu.VMEM((BPW, DEG), jnp.int32),
            pltpu.VMEM((BPW * DEG,), jnp.int32),
            pltpu.SemaphoreType.DMA,
        ],
    )
    def k(nodes_hbm, neigh_hbm, out_hbm, idx_v, nidx_v, flat_v, sem):
        wid = lax.axis_index("s") * NC + lax.axis_index("c")
        base = wid * BPW
        pltpu.sync_copy(nodes_hbm.at[pl.ds(base, BPW)], idx_v)
        pltpu.async_copy(neigh_hbm.at[idx_v], nidx_v, sem).wait()

        @pl.loop(0, BPW)
        def _(i):
            for j in range(DEG // L):
                flat_v[pl.ds(i * DEG + j * L, L)] = nidx_v[i, pl.ds(j * L, L)]

        pltpu.sync_copy(flat_v, out_hbm.at[pl.ds(base * DEG, BPW * DEG)])

    return k(nodes, neighbors)


def _sc_gather_agg(nodes, flat_idx, q_tab, p_tab):
    B, = nodes.shape
    DEG = flat_idx.shape[0] // B
    D = q_tab.shape[1]
    BPW = B // NW
    CH = 4
    NCHUNK = BPW // CH
    mesh = plsc.VectorSubcoreMesh(core_axis_name="c", subcore_axis_name="s")

    @functools.partial(
        pl.kernel,
        mesh=mesh,
        out_type=jax.ShapeDtypeStruct((B, D), jnp.float32),
        compiler_params=pltpu.CompilerParams(use_tc_tiling_on_sc=False,
                                             needs_layout_passes=False),
        scratch_types=[
            pltpu.VMEM((BPW,), jnp.int32),
            pltpu.VMEM((BPW * DEG,), jnp.int32),
            pltpu.VMEM((BPW, D), jnp.float32),
            pltpu.VMEM((CH * DEG, D), jnp.bfloat16),
            pltpu.VMEM((CH * DEG, D), jnp.bfloat16),
            pltpu.VMEM((BPW, D), jnp.float32),
            pltpu.SemaphoreType.DMA,
            pltpu.SemaphoreType.DMA,
            pltpu.SemaphoreType.DMA,
        ],
    )
    def k(nodes_hbm, flat_hbm, q_hbm, p_hbm, out_hbm,
          idx_v, flat_v, q_v, buf0, buf1, out_v, sem0, sem1, semq):
        wid = lax.axis_index("s") * NC + lax.axis_index("c")
        base = wid * BPW
        pltpu.sync_copy(nodes_hbm.at[pl.ds(base, BPW)], idx_v)
        pltpu.async_copy(q_hbm.at[idx_v], q_v, semq)
        pltpu.sync_copy(flat_hbm.at[pl.ds(base * DEG, BPW * DEG)], flat_v)

        def issue(c, buf, sem):
            pltpu.async_copy(
                p_hbm.at[flat_v.at[pl.ds(c * (CH * DEG), CH * DEG)]], buf, sem)

        def drain(buf, sem):
            pltpu.make_async_copy(
                p_hbm.at[flat_v.at[pl.ds(0, CH * DEG)]], buf, sem).wait()

        def accum(c, buf):
            @pl.loop(0, CH)
            def _(rr):
                row = c * CH + rr
                for g in range(D // (2 * L)):
                    sl = pl.ds(g * 2 * L, 2 * L)
                    accs = [buf[rr * DEG + t, sl] for t in range(4)]
                    for j in range(4, DEG):
                        accs[j % 4] = accs[j % 4] + buf[rr * DEG + j, sl]
                    acc = (accs[0] + accs[1]) + (accs[2] + accs[3])
                    lo, hi = plsc.unpack(acc, format=plsc.PackFormat.INTERLEAVED)
                    sl_lo = pl.ds(g * 2 * L, L)
                    sl_hi = pl.ds(g * 2 * L + L, L)
                    out_v[row, sl_lo] = jnp.maximum(q_v[row, sl_lo] + lo, 0.0)
                    out_v[row, sl_hi] = jnp.maximum(q_v[row, sl_hi] + hi, 0.0)

        issue(0, buf0, sem0)
        pltpu.make_async_copy(q_hbm.at[idx_v], q_v, semq).wait()

        @pl.loop(0, NCHUNK, step=2)
        def _(c):
            issue(c + 1, buf1, sem1)
            drain(buf0, sem0)
            accum(c, buf0)

            @pl.when(c + 2 < NCHUNK)
            def _():
                issue(c + 2, buf0, sem0)

            drain(buf1, sem1)
            accum(c + 1, buf1)

        pltpu.sync_copy(out_v, out_hbm.at[pl.ds(base, BPW)])

    return k(nodes, flat_idx, q_tab, p_tab)


def kernel(nodes, neighbors, u2e_weight, base_weight, W1, b1):
    flat_idx = _sc_index(nodes, neighbors)
    q_tab, p_tab = _project(u2e_weight, base_weight, W1, b1)
    return _sc_gather_agg(nodes, flat_idx, q_tab, p_tab)


# speedup vs baseline: 1.1898x; 1.0602x over previous
import functools

import jax
import jax.numpy as jnp
import numpy as np
from jax import lax
from jax.experimental import pallas as pl
from jax.experimental.pallas import tpu as pltpu
from jax.experimental.pallas import tpu_sc as plsc

NC = 2
NS = 16
NW = NC * NS
L = 16


def _interleave_perm(d):
    perm = np.zeros(d, dtype=np.int32)
    for g in range(d // 32):
        for i in range(16):
            perm[g * 32 + 2 * i] = g * 32 + i
            perm[g * 32 + 2 * i + 1] = g * 32 + 16 + i
    return perm


def _project(u2e, base, W1, b1):
    N, D = u2e.shape
    deg_inv = 1.0 / 32.0
    Wa = W1[0:D]
    Wb = W1[D:2 * D][:, _interleave_perm(D)]
    Wc = W1[2 * D:3 * D]
    b1_2d = b1.reshape(1, D)

    BLK = 2000
    assert N % BLK == 0

    def body(u_ref, c_ref, wa_ref, wb_ref, wc_ref, b1_ref, q_ref, p_ref):
        u = u_ref[...]
        q_ref[...] = (
            jnp.dot(u, wa_ref[...], preferred_element_type=jnp.float32,
                    precision=lax.Precision.DEFAULT)
            + jnp.dot(c_ref[...], wc_ref[...], preferred_element_type=jnp.float32,
                      precision=lax.Precision.DEFAULT)
            + b1_ref[...]
        )
        p_ref[...] = (jnp.dot(u, wb_ref[...], preferred_element_type=jnp.float32,
                              precision=lax.Precision.DEFAULT)
                      * deg_inv).astype(jnp.bfloat16)

    grid = (N // BLK,)
    return pl.pallas_call(
        body,
        grid=grid,
        in_specs=[
            pl.BlockSpec((BLK, D), lambda i: (i, 0)),
            pl.BlockSpec((BLK, D), lambda i: (i, 0)),
            pl.BlockSpec((D, D), lambda i: (0, 0)),
            pl.BlockSpec((D, D), lambda i: (0, 0)),
            pl.BlockSpec((D, D), lambda i: (0, 0)),
            pl.BlockSpec((1, D), lambda i: (0, 0)),
        ],
        out_specs=[
            pl.BlockSpec((BLK, D), lambda i: (i, 0)),
            pl.BlockSpec((BLK, D), lambda i: (i, 0)),
        ],
        out_shape=[
            jax.ShapeDtypeStruct((N, D), jnp.float32),
            jax.ShapeDtypeStruct((N, D), jnp.bfloat16),
        ],
    )(u2e, base, Wa, Wb, Wc, b1_2d)


def _sc_index(nodes, neighbors):
    B, = nodes.shape
    N, DEG = neighbors.shape
    BPW = B // NW
    mesh = plsc.VectorSubcoreMesh(core_axis_name="c", subcore_axis_name="s")

    @functools.partial(
        pl.kernel,
        mesh=mesh,
        out_type=jax.ShapeDtypeStruct((B * DEG,), jnp.int32),
        compiler_params=pltpu.CompilerParams(use_tc_tiling_on_sc=False,
                                             needs_layout_passes=False),
        scratch_types=[
            pltpu.VMEM((BPW,), jnp.int32),
            pltpu.VMEM((BPW, DEG), jnp.int32),
            pltpu.VMEM((BPW * DEG,), jnp.int32),
            pltpu.SemaphoreType.DMA,
        ],
    )
    def k(nodes_hbm, neigh_hbm, out_hbm, idx_v, nidx_v, flat_v, sem):
        wid = lax.axis_index("s") * NC + lax.axis_index("c")
        base = wid * BPW
        pltpu.sync_copy(nodes_hbm.at[pl.ds(base, BPW)], idx_v)
        pltpu.async_copy(neigh_hbm.at[idx_v], nidx_v, sem).wait()

        @pl.loop(0, BPW)
        def _(i):
            for j in range(DEG // L):
                flat_v[pl.ds(i * DEG + j * L, L)] = nidx_v[i, pl.ds(j * L, L)]

        pltpu.sync_copy(flat_v, out_hbm.at[pl.ds(base * DEG, BPW * DEG)])

    return k(nodes, neighbors)


def _sc_gather_agg(nodes, flat_idx, q_tab, p_tab):
    B, = nodes.shape
    DEG = flat_idx.shape[0] // B
    D = q_tab.shape[1]
    BPW = B // NW
    CH = 8
    NCHUNK = BPW // CH
    mesh = plsc.VectorSubcoreMesh(core_axis_name="c", subcore_axis_name="s")

    @functools.partial(
        pl.kernel,
        mesh=mesh,
        out_type=jax.ShapeDtypeStruct((B, D), jnp.float32),
        compiler_params=pltpu.CompilerParams(use_tc_tiling_on_sc=False,
                                             needs_layout_passes=False),
        scratch_types=[
            pltpu.VMEM((BPW,), jnp.int32),
            pltpu.VMEM((BPW * DEG,), jnp.int32),
            pltpu.VMEM((BPW, D), jnp.float32),
            pltpu.VMEM((CH * DEG, D), jnp.bfloat16),
            pltpu.VMEM((CH * DEG, D), jnp.bfloat16),
            pltpu.VMEM((BPW, D), jnp.float32),
            pltpu.SemaphoreType.DMA,
            pltpu.SemaphoreType.DMA,
            pltpu.SemaphoreType.DMA,
        ],
    )
    def k(nodes_hbm, flat_hbm, q_hbm, p_hbm, out_hbm,
          idx_v, flat_v, q_v, buf0, buf1, out_v, sem0, sem1, semq):
        wid = lax.axis_index("s") * NC + lax.axis_index("c")
        base = wid * BPW
        pltpu.sync_copy(nodes_hbm.at[pl.ds(base, BPW)], idx_v)
        pltpu.async_copy(q_hbm.at[idx_v], q_v, semq)
        pltpu.sync_copy(flat_hbm.at[pl.ds(base * DEG, BPW * DEG)], flat_v)

        def issue(c, buf, sem):
            pltpu.async_copy(
                p_hbm.at[flat_v.at[pl.ds(c * (CH * DEG), CH * DEG)]], buf, sem)

        def drain(buf, sem):
            pltpu.make_async_copy(
                p_hbm.at[flat_v.at[pl.ds(0, CH * DEG)]], buf, sem).wait()

        def accum(c, buf):
            @pl.loop(0, CH)
            def _(rr):
                row = c * CH + rr
                for g in range(D // (2 * L)):
                    sl = pl.ds(g * 2 * L, 2 * L)
                    accs = [buf[rr * DEG + t, sl] for t in range(4)]
                    for j in range(4, DEG):
                        accs[j % 4] = accs[j % 4] + buf[rr * DEG + j, sl]
                    acc = (accs[0] + accs[1]) + (accs[2] + accs[3])
                    lo, hi = plsc.unpack(acc, format=plsc.PackFormat.INTERLEAVED)
                    sl_lo = pl.ds(g * 2 * L, L)
                    sl_hi = pl.ds(g * 2 * L + L, L)
                    out_v[row, sl_lo] = jnp.maximum(q_v[row, sl_lo] + lo, 0.0)
                    out_v[row, sl_hi] = jnp.maximum(q_v[row, sl_hi] + hi, 0.0)

        issue(0, buf0, sem0)
        pltpu.make_async_copy(q_hbm.at[idx_v], q_v, semq).wait()

        @pl.loop(0, NCHUNK, step=2)
        def _(c):
            issue(c + 1, buf1, sem1)
            drain(buf0, sem0)
            accum(c, buf0)

            @pl.when(c + 2 < NCHUNK)
            def _():
                issue(c + 2, buf0, sem0)

            drain(buf1, sem1)
            accum(c + 1, buf1)

        pltpu.sync_copy(out_v, out_hbm.at[pl.ds(base, BPW)])

    return k(nodes, flat_idx, q_tab, p_tab)


def kernel(nodes, neighbors, u2e_weight, base_weight, W1, b1):
    flat_idx = _sc_index(nodes, neighbors)
    q_tab, p_tab = _project(u2e_weight, base_weight, W1, b1)
    return _sc_gather_agg(nodes, flat_idx, q_tab, p_tab)


# speedup vs baseline: 1.1964x; 1.0055x over previous
import functools

import jax
import jax.numpy as jnp
import numpy as np
from jax import lax
from jax.experimental import pallas as pl
from jax.experimental.pallas import tpu as pltpu
from jax.experimental.pallas import tpu_sc as plsc

NC = 2
NS = 16
NW = NC * NS
L = 16


def _interleave_perm(d):
    perm = np.zeros(d, dtype=np.int32)
    for g in range(d // 32):
        for i in range(16):
            perm[g * 32 + 2 * i] = g * 32 + i
            perm[g * 32 + 2 * i + 1] = g * 32 + 16 + i
    return perm


def _project(u2e, base, W1, b1):
    N, D = u2e.shape
    deg_inv = 1.0 / 32.0
    Wa = W1[0:D]
    Wb = W1[D:2 * D][:, _interleave_perm(D)]
    Wc = W1[2 * D:3 * D]
    b1_2d = b1.reshape(1, D)

    BLK = 2000
    assert N % BLK == 0

    def body(u_ref, c_ref, wa_ref, wb_ref, wc_ref, b1_ref, q_ref, p_ref):
        u = u_ref[...]
        q_ref[...] = (
            jnp.dot(u, wa_ref[...], preferred_element_type=jnp.float32,
                    precision=lax.Precision.DEFAULT)
            + jnp.dot(c_ref[...], wc_ref[...], preferred_element_type=jnp.float32,
                      precision=lax.Precision.DEFAULT)
            + b1_ref[...]
        )
        p_ref[...] = (jnp.dot(u, wb_ref[...], preferred_element_type=jnp.float32,
                              precision=lax.Precision.DEFAULT)
                      * deg_inv).astype(jnp.bfloat16)

    grid = (N // BLK,)
    return pl.pallas_call(
        body,
        grid=grid,
        in_specs=[
            pl.BlockSpec((BLK, D), lambda i: (i, 0)),
            pl.BlockSpec((BLK, D), lambda i: (i, 0)),
            pl.BlockSpec((D, D), lambda i: (0, 0)),
            pl.BlockSpec((D, D), lambda i: (0, 0)),
            pl.BlockSpec((D, D), lambda i: (0, 0)),
            pl.BlockSpec((1, D), lambda i: (0, 0)),
        ],
        out_specs=[
            pl.BlockSpec((BLK, D), lambda i: (i, 0)),
            pl.BlockSpec((BLK, D), lambda i: (i, 0)),
        ],
        out_shape=[
            jax.ShapeDtypeStruct((N, D), jnp.float32),
            jax.ShapeDtypeStruct((N, D), jnp.bfloat16),
        ],
    )(u2e, base, Wa, Wb, Wc, b1_2d)


def _sc_index(nodes, neighbors):
    B, = nodes.shape
    N, DEG = neighbors.shape
    BPW = B // NW
    mesh = plsc.VectorSubcoreMesh(core_axis_name="c", subcore_axis_name="s")

    @functools.partial(
        pl.kernel,
        mesh=mesh,
        out_type=jax.ShapeDtypeStruct((B * DEG,), jnp.int32),
        compiler_params=pltpu.CompilerParams(use_tc_tiling_on_sc=False,
                                             needs_layout_passes=False),
        scratch_types=[
            pltpu.VMEM((BPW,), jnp.int32),
            pltpu.VMEM((BPW, DEG), jnp.int32),
            pltpu.VMEM((BPW * DEG,), jnp.int32),
            pltpu.SemaphoreType.DMA,
        ],
    )
    def k(nodes_hbm, neigh_hbm, out_hbm, idx_v, nidx_v, flat_v, sem):
        wid = lax.axis_index("s") * NC + lax.axis_index("c")
        base = wid * BPW
        pltpu.sync_copy(nodes_hbm.at[pl.ds(base, BPW)], idx_v)
        pltpu.async_copy(neigh_hbm.at[idx_v], nidx_v, sem).wait()

        @pl.loop(0, BPW)
        def _(i):
            for j in range(DEG // L):
                flat_v[pl.ds(i * DEG + j * L, L)] = nidx_v[i, pl.ds(j * L, L)]

        pltpu.sync_copy(flat_v, out_hbm.at[pl.ds(base * DEG, BPW * DEG)])

    return k(nodes, neighbors)


def _sc_gather_agg(nodes, flat_idx, q_tab, p_tab):
    B, = nodes.shape
    DEG = flat_idx.shape[0] // B
    D = q_tab.shape[1]
    BPW = B // NW
    CH = 16
    NCHUNK = BPW // CH
    mesh = plsc.VectorSubcoreMesh(core_axis_name="c", subcore_axis_name="s")

    @functools.partial(
        pl.kernel,
        mesh=mesh,
        out_type=jax.ShapeDtypeStruct((B, D), jnp.float32),
        compiler_params=pltpu.CompilerParams(use_tc_tiling_on_sc=False,
                                             needs_layout_passes=False),
        scratch_types=[
            pltpu.VMEM((BPW,), jnp.int32),
            pltpu.VMEM((BPW * DEG,), jnp.int32),
            pltpu.VMEM((BPW, D), jnp.float32),
            pltpu.VMEM((CH * DEG, D), jnp.bfloat16),
            pltpu.VMEM((CH * DEG, D), jnp.bfloat16),
            pltpu.VMEM((BPW, D), jnp.float32),
            pltpu.SemaphoreType.DMA,
            pltpu.SemaphoreType.DMA,
            pltpu.SemaphoreType.DMA,
        ],
    )
    def k(nodes_hbm, flat_hbm, q_hbm, p_hbm, out_hbm,
          idx_v, flat_v, q_v, buf0, buf1, out_v, sem0, sem1, semq):
        wid = lax.axis_index("s") * NC + lax.axis_index("c")
        base = wid * BPW
        pltpu.sync_copy(nodes_hbm.at[pl.ds(base, BPW)], idx_v)
        pltpu.async_copy(q_hbm.at[idx_v], q_v, semq)
        pltpu.sync_copy(flat_hbm.at[pl.ds(base * DEG, BPW * DEG)], flat_v)

        def issue(c, buf, sem):
            pltpu.async_copy(
                p_hbm.at[flat_v.at[pl.ds(c * (CH * DEG), CH * DEG)]], buf, sem)

        def drain(buf, sem):
            pltpu.make_async_copy(
                p_hbm.at[flat_v.at[pl.ds(0, CH * DEG)]], buf, sem).wait()

        def accum(c, buf):
            @pl.loop(0, CH)
            def _(rr):
                row = c * CH + rr
                for g in range(D // (2 * L)):
                    sl = pl.ds(g * 2 * L, 2 * L)
                    accs = [buf[rr * DEG + t, sl] for t in range(4)]
                    for j in range(4, DEG):
                        accs[j % 4] = accs[j % 4] + buf[rr * DEG + j, sl]
                    acc = (accs[0] + accs[1]) + (accs[2] + accs[3])
                    lo, hi = plsc.unpack(acc, format=plsc.PackFormat.INTERLEAVED)
                    sl_lo = pl.ds(g * 2 * L, L)
                    sl_hi = pl.ds(g * 2 * L + L, L)
                    out_v[row, sl_lo] = jnp.maximum(q_v[row, sl_lo] + lo, 0.0)
                    out_v[row, sl_hi] = jnp.maximum(q_v[row, sl_hi] + hi, 0.0)

        issue(0, buf0, sem0)
        pltpu.make_async_copy(q_hbm.at[idx_v], q_v, semq).wait()

        @pl.loop(0, NCHUNK, step=2)
        def _(c):
            issue(c + 1, buf1, sem1)
            drain(buf0, sem0)
            accum(c, buf0)

            @pl.when(c + 2 < NCHUNK)
            def _():
                issue(c + 2, buf0, sem0)

            drain(buf1, sem1)
            accum(c + 1, buf1)

        pltpu.sync_copy(out_v, out_hbm.at[pl.ds(base, BPW)])

    return k(nodes, flat_idx, q_tab, p_tab)


def kernel(nodes, neighbors, u2e_weight, base_weight, W1, b1):
    flat_idx = _sc_index(nodes, neighbors)
    q_tab, p_tab = _project(u2e_weight, base_weight, W1, b1)
    return _sc_gather_agg(nodes, flat_idx, q_tab, p_tab)


# speedup vs baseline: 1.2028x; 1.0054x over previous
import functools

import jax
import jax.numpy as jnp
import numpy as np
from jax import lax
from jax.experimental import pallas as pl
from jax.experimental.pallas import tpu as pltpu
from jax.experimental.pallas import tpu_sc as plsc

NC = 2
NS = 16
NW = NC * NS
L = 16


def _interleave_perm(d):
    perm = np.zeros(d, dtype=np.int32)
    for g in range(d // 32):
        for i in range(16):
            perm[g * 32 + 2 * i] = g * 32 + i
            perm[g * 32 + 2 * i + 1] = g * 32 + 16 + i
    return perm


def _project(u2e, base, W1, b1):
    N, D = u2e.shape
    deg_inv = 1.0 / 32.0
    Wa = W1[0:D]
    Wb = W1[D:2 * D][:, _interleave_perm(D)]
    Wc = W1[2 * D:3 * D]
    b1_2d = b1.reshape(1, D)

    BLK = 2000
    assert N % BLK == 0

    def body(u_ref, c_ref, wa_ref, wb_ref, wc_ref, b1_ref, q_ref, p_ref):
        u = u_ref[...]
        q_ref[...] = (
            jnp.dot(u, wa_ref[...], preferred_element_type=jnp.float32,
                    precision=lax.Precision.DEFAULT)
            + jnp.dot(c_ref[...], wc_ref[...], preferred_element_type=jnp.float32,
                      precision=lax.Precision.DEFAULT)
            + b1_ref[...]
        )
        p_ref[...] = (jnp.dot(u, wb_ref[...], preferred_element_type=jnp.float32,
                              precision=lax.Precision.DEFAULT)
                      * deg_inv).astype(jnp.bfloat16)

    grid = (N // BLK,)
    return pl.pallas_call(
        body,
        grid=grid,
        in_specs=[
            pl.BlockSpec((BLK, D), lambda i: (i, 0)),
            pl.BlockSpec((BLK, D), lambda i: (i, 0)),
            pl.BlockSpec((D, D), lambda i: (0, 0)),
            pl.BlockSpec((D, D), lambda i: (0, 0)),
            pl.BlockSpec((D, D), lambda i: (0, 0)),
            pl.BlockSpec((1, D), lambda i: (0, 0)),
        ],
        out_specs=[
            pl.BlockSpec((BLK, D), lambda i: (i, 0)),
            pl.BlockSpec((BLK, D), lambda i: (i, 0)),
        ],
        out_shape=[
            jax.ShapeDtypeStruct((N, D), jnp.float32),
            jax.ShapeDtypeStruct((N, D), jnp.bfloat16),
        ],
    )(u2e, base, Wa, Wb, Wc, b1_2d)


def _sc_index(nodes, neighbors):
    B, = nodes.shape
    N, DEG = neighbors.shape
    BPW = B // NW
    mesh = plsc.VectorSubcoreMesh(core_axis_name="c", subcore_axis_name="s")

    @functools.partial(
        pl.kernel,
        mesh=mesh,
        out_type=jax.ShapeDtypeStruct((B * DEG,), jnp.int32),
        compiler_params=pltpu.CompilerParams(use_tc_tiling_on_sc=False,
                                             needs_layout_passes=False),
        scratch_types=[
            pltpu.VMEM((BPW,), jnp.int32),
            pltpu.VMEM((BPW, DEG), jnp.int32),
            pltpu.VMEM((BPW * DEG,), jnp.int32),
            pltpu.SemaphoreType.DMA,
        ],
    )
    def k(nodes_hbm, neigh_hbm, out_hbm, idx_v, nidx_v, flat_v, sem):
        wid = lax.axis_index("s") * NC + lax.axis_index("c")
        base = wid * BPW
        pltpu.sync_copy(nodes_hbm.at[pl.ds(base, BPW)], idx_v)
        pltpu.async_copy(neigh_hbm.at[idx_v], nidx_v, sem).wait()

        @pl.loop(0, BPW)
        def _(i):
            for j in range(DEG // L):
                flat_v[pl.ds(i * DEG + j * L, L)] = nidx_v[i, pl.ds(j * L, L)]

        pltpu.sync_copy(flat_v, out_hbm.at[pl.ds(base * DEG, BPW * DEG)])

    return k(nodes, neighbors)


def _sc_gather_agg(nodes, flat_idx, q_tab, p_tab):
    B, = nodes.shape
    DEG = flat_idx.shape[0] // B
    D = q_tab.shape[1]
    BPW = B // NW
    CH = 8
    NCHUNK = BPW // CH
    mesh = plsc.VectorSubcoreMesh(core_axis_name="c", subcore_axis_name="s")

    @functools.partial(
        pl.kernel,
        mesh=mesh,
        out_type=jax.ShapeDtypeStruct((B, D), jnp.float32),
        compiler_params=pltpu.CompilerParams(use_tc_tiling_on_sc=False,
                                             needs_layout_passes=False),
        scratch_types=[
            pltpu.VMEM((BPW,), jnp.int32),
            pltpu.VMEM((BPW * DEG,), jnp.int32),
            pltpu.VMEM((BPW, D), jnp.float32),
            pltpu.VMEM((CH * DEG, D), jnp.bfloat16),
            pltpu.VMEM((CH * DEG, D), jnp.bfloat16),
            pltpu.VMEM((CH * DEG, D), jnp.bfloat16),
            pltpu.VMEM((CH * DEG, D), jnp.bfloat16),
            pltpu.VMEM((BPW, D), jnp.float32),
            pltpu.SemaphoreType.DMA,
            pltpu.SemaphoreType.DMA,
            pltpu.SemaphoreType.DMA,
            pltpu.SemaphoreType.DMA,
            pltpu.SemaphoreType.DMA,
        ],
    )
    def k(nodes_hbm, flat_hbm, q_hbm, p_hbm, out_hbm,
          idx_v, flat_v, q_v, buf0, buf1, buf2, buf3, out_v,
          sem0, sem1, sem2, sem3, semq):
        wid = lax.axis_index("s") * NC + lax.axis_index("c")
        base = wid * BPW
        pltpu.sync_copy(nodes_hbm.at[pl.ds(base, BPW)], idx_v)
        pltpu.async_copy(q_hbm.at[idx_v], q_v, semq)
        pltpu.sync_copy(flat_hbm.at[pl.ds(base * DEG, BPW * DEG)], flat_v)

        def issue(c, buf, sem):
            pltpu.async_copy(
                p_hbm.at[flat_v.at[pl.ds(c * (CH * DEG), CH * DEG)]], buf, sem)

        def drain(buf, sem):
            pltpu.make_async_copy(
                p_hbm.at[flat_v.at[pl.ds(0, CH * DEG)]], buf, sem).wait()

        def accum(c, buf):
            @pl.loop(0, CH)
            def _(rr):
                row = c * CH + rr
                for g in range(D // (2 * L)):
                    sl = pl.ds(g * 2 * L, 2 * L)
                    accs = [buf[rr * DEG + t, sl] for t in range(4)]
                    for j in range(4, DEG):
                        accs[j % 4] = accs[j % 4] + buf[rr * DEG + j, sl]
                    acc = (accs[0] + accs[1]) + (accs[2] + accs[3])
                    lo, hi = plsc.unpack(acc, format=plsc.PackFormat.INTERLEAVED)
                    sl_lo = pl.ds(g * 2 * L, L)
                    sl_hi = pl.ds(g * 2 * L + L, L)
                    out_v[row, sl_lo] = jnp.maximum(q_v[row, sl_lo] + lo, 0.0)
                    out_v[row, sl_hi] = jnp.maximum(q_v[row, sl_hi] + hi, 0.0)

        bufs = (buf0, buf1, buf2, buf3)
        sems = (sem0, sem1, sem2, sem3)
        NBUF = 4
        issue(0, bufs[0], sems[0])
        issue(1, bufs[1], sems[1])
        issue(2, bufs[2], sems[2])
        pltpu.make_async_copy(q_hbm.at[idx_v], q_v, semq).wait()

        @pl.loop(0, NCHUNK, step=NBUF)
        def _(c):
            for t in range(NBUF):
                drain(bufs[t], sems[t])
                accum(c + t, bufs[t])

                @pl.when(c + t + NBUF - 1 < NCHUNK)
                def _():
                    issue(c + t + NBUF - 1, bufs[(t + NBUF - 1) % NBUF],
                          sems[(t + NBUF - 1) % NBUF])

        pltpu.sync_copy(out_v, out_hbm.at[pl.ds(base, BPW)])

    return k(nodes, flat_idx, q_tab, p_tab)


def kernel(nodes, neighbors, u2e_weight, base_weight, W1, b1):
    flat_idx = _sc_index(nodes, neighbors)
    q_tab, p_tab = _project(u2e_weight, base_weight, W1, b1)
    return _sc_gather_agg(nodes, flat_idx, q_tab, p_tab)
